# Initial kernel scaffold; baseline (speedup 1.0000x reference)
import jax
import jax.numpy as jnp
from jax.experimental import pallas as pl


def kernel(state, edge_index, edge_attr, mask, W1, b1, gamma, beta, W2, b2):
    raise NotImplementedError("write your pallas kernel here")


# speedup vs baseline: 26.3500x; 26.3500x over previous
import functools

import jax
import jax.numpy as jnp
from jax import lax
from jax.experimental import pallas as pl
from jax.experimental.pallas import tpu as pltpu
from jax.experimental.pallas import tpu_sc as plsc

N = 10000
NPAD = 10240
E = 320000
F = 128
H = 512
OUT = 2
NC, NS = 2, 16
NW = NC * NS
EPT = E // NW
CH, CW = 80, 125
ROWS_PER_TILE = NPAD // NS

_MESH = dict(core_axis_name="c", subcore_axis_name="s", num_cores=NC,
             num_subcores=NS)


def _deg_body(dst_hbm, ones_hbm, zeros_hbm, out_hbm, didx_v, ones_v, deg_sh):
    c = lax.axis_index("c")
    s = lax.axis_index("s")
    wid = c * NS + s
    pltpu.sync_copy(dst_hbm.at[wid], didx_v)
    pltpu.sync_copy(ones_hbm, ones_v)
    pltpu.sync_copy(zeros_hbm, deg_sh.at[pl.ds(s * ROWS_PER_TILE,
                                               ROWS_PER_TILE)])
    plsc.subcore_barrier()

    def body(j, carry):
        pltpu.sync_copy(ones_v.at[pl.ds(0, CW)], deg_sh.at[didx_v.at[j]],
                        add=True)
        return carry

    lax.fori_loop(0, CH, body, 0)
    plsc.subcore_barrier()
    sl = pl.ds(s * ROWS_PER_TILE, ROWS_PER_TILE)
    pltpu.sync_copy(deg_sh.at[sl], out_hbm.at[c, sl])


def _deg_sc(dst3, ones128, zeros640):
    k = functools.partial(
        pl.kernel,
        out_type=jax.ShapeDtypeStruct((NC, NPAD), jnp.float32),
        mesh=plsc.VectorSubcoreMesh(**_MESH),
        scratch_types=[
            pltpu.VMEM((CH, CW), jnp.int32),
            pltpu.VMEM((128,), jnp.float32),
            pltpu.VMEM_SHARED((NPAD,), jnp.float32),
        ],
    )(_deg_body)
    return k(dst3, ones128, zeros640)


def _agg_body(y_hbm, src_hbm, dst_hbm, zeros_hbm, out_hbm, sidx_v, didx_v,
              gbuf, z_sh):
    c = lax.axis_index("c")
    s = lax.axis_index("s")
    wid = c * NS + s
    pltpu.sync_copy(src_hbm.at[wid], sidx_v)
    pltpu.sync_copy(dst_hbm.at[wid], didx_v)
    rsl = pl.ds(s * ROWS_PER_TILE, ROWS_PER_TILE)
    pltpu.sync_copy(zeros_hbm, z_sh.at[rsl, :])
    plsc.subcore_barrier()

    def body(j, carry):
        pltpu.sync_copy(y_hbm.at[sidx_v.at[j]], gbuf)
        pltpu.sync_copy(gbuf, z_sh.at[didx_v.at[j]], add=True)
        return carry

    lax.fori_loop(0, CH, body, 0)
    plsc.subcore_barrier()
    pltpu.sync_copy(z_sh.at[rsl, :], out_hbm.at[c, rsl, :])


def _agg_sc(y, src3, dst3, zeros_blk):
    k = functools.partial(
        pl.kernel,
        out_type=jax.ShapeDtypeStruct((NC, NPAD, F), jnp.float32),
        mesh=plsc.VectorSubcoreMesh(**_MESH),
        scratch_types=[
            pltpu.VMEM((CH, CW), jnp.int32),
            pltpu.VMEM((CH, CW), jnp.int32),
            pltpu.VMEM((CW, F), jnp.float32),
            pltpu.VMEM_SHARED((NPAD, F), jnp.float32),
        ],
    )(_agg_body)
    return k(y, src3, dst3, zeros_blk)


def _scale_body(x_ref, d_ref, y_ref, dinv_ref):
    inv = lax.rsqrt(d_ref[...] + 1.0)
    y_ref[...] = x_ref[...] * inv
    dinv_ref[...] = inv


def _scale_tc(x_pad, degcol):
    return pl.pallas_call(
        _scale_body,
        grid=(NPAD // 128,),
        in_specs=[
            pl.BlockSpec((128, F), lambda i: (i, 0)),
            pl.BlockSpec((128, 1), lambda i: (i, 0)),
        ],
        out_specs=[
            pl.BlockSpec((128, F), lambda i: (i, 0)),
            pl.BlockSpec((128, 1), lambda i: (i, 0)),
        ],
        out_shape=[
            jax.ShapeDtypeStruct((NPAD, F), jnp.float32),
            jax.ShapeDtypeStruct((NPAD, 1), jnp.float32),
        ],
    )(x_pad, degcol)


def _stats_body(z_ref, y_ref, dinv_ref, G_ref, m_ref, agg_ref):
    i = pl.program_id(0)
    z = z_ref[...]
    a = (z[0] + z[1] + y_ref[...]) * dinv_ref[...]
    agg_ref[...] = a

    @pl.when(i == 0)
    def _():
        G_ref[...] = jnp.zeros_like(G_ref)
        m_ref[...] = jnp.zeros_like(m_ref)

    G_ref[...] += lax.dot_general(
        a, a, (((0,), (0,)), ((), ())),
        preferred_element_type=jnp.float32,
        precision=lax.Precision.HIGHEST)
    m_ref[...] += jnp.sum(a, axis=0, keepdims=True)


def _stats_tc(z, y, dinv):
    return pl.pallas_call(
        _stats_body,
        grid=(NPAD // 128,),
        in_specs=[
            pl.BlockSpec((NC, 128, F), lambda i: (0, i, 0)),
            pl.BlockSpec((128, F), lambda i: (i, 0)),
            pl.BlockSpec((128, 1), lambda i: (i, 0)),
        ],
        out_specs=[
            pl.BlockSpec((F, F), lambda i: (0, 0)),
            pl.BlockSpec((1, F), lambda i: (0, 0)),
            pl.BlockSpec((128, F), lambda i: (i, 0)),
        ],
        out_shape=[
            jax.ShapeDtypeStruct((F, F), jnp.float32),
            jax.ShapeDtypeStruct((1, F), jnp.float32),
            jax.ShapeDtypeStruct((NPAD, F), jnp.float32),
        ],
    )(z, y, dinv)


def _finalize_body(G_ref, m_ref, W1_ref, g_ref, be_ref, W2_ref, b2_ref,
                   M_ref, c_ref):
    invn = 1.0 / N
    W1 = W1_ref[...]
    mn = m_ref[...] * invn
    mh0 = lax.dot_general(mn, W1, (((1,), (0,)), ((), ())),
                          preferred_element_type=jnp.float32,
                          precision=lax.Precision.HIGHEST)
    GW = lax.dot_general(G_ref[...] * invn, W1,
                         (((1,), (0,)), ((), ())),
                         preferred_element_type=jnp.float32,
                         precision=lax.Precision.HIGHEST)
    varh = jnp.sum(GW * W1, axis=0, keepdims=True) - mh0 * mh0
    sv = g_ref[...] * lax.rsqrt(varh + 1e-5)
    M_ref[...] = lax.dot_general(W1 * sv, W2_ref[...],
                                 (((1,), (0,)), ((), ())),
                                 preferred_element_type=jnp.float32,
                                 precision=lax.Precision.HIGHEST)
    c_ref[...] = lax.dot_general(be_ref[...] - mh0 * sv, W2_ref[...],
                                 (((1,), (0,)), ((), ())),
                                 preferred_element_type=jnp.float32,
                                 precision=lax.Precision.HIGHEST) + b2_ref[...]


def _finalize_tc(G, m, W1, gamma2, beta2, W2, b22):
    return pl.pallas_call(
        _finalize_body,
        out_shape=[
            jax.ShapeDtypeStruct((F, OUT), jnp.float32),
            jax.ShapeDtypeStruct((1, OUT), jnp.float32),
        ],
    )(G, m, W1, gamma2, beta2, W2, b22)


def _head_body(agg_ref, M_ref, c_ref, mk_ref, o_ref):
    l = lax.dot_general(agg_ref[...], M_ref[...], (((1,), (0,)), ((), ())),
                        preferred_element_type=jnp.float32,
                        precision=lax.Precision.HIGHEST) + c_ref[...]
    r = jnp.maximum(l, 0.0) * mk_ref[...]
    mx = jnp.max(r, axis=1, keepdims=True)
    e = jnp.exp(r - mx)
    o_ref[...] = e / jnp.sum(e, axis=1, keepdims=True)


def _head_tc(agg, M, c, maskcol):
    return pl.pallas_call(
        _head_body,
        grid=(NPAD // 128,),
        in_specs=[
            pl.BlockSpec((128, F), lambda i: (i, 0)),
            pl.BlockSpec((F, OUT), lambda i: (0, 0)),
            pl.BlockSpec((1, OUT), lambda i: (0, 0)),
            pl.BlockSpec((128, 1), lambda i: (i, 0)),
        ],
        out_specs=pl.BlockSpec((128, OUT), lambda i: (i, 0)),
        out_shape=jax.ShapeDtypeStruct((NPAD, OUT), jnp.float32),
    )(agg, M, c, maskcol)


def kernel(state, edge_index, edge_attr, mask, W1, b1, gamma, beta, W2, b2):
    del edge_attr, b1
    x = state.reshape(N, F)
    x_pad = jnp.pad(x, ((0, NPAD - N), (0, 0)))
    src3 = edge_index[0].reshape(NW, CH, CW)
    dst3 = edge_index[1].reshape(NW, CH, CW)
    ones128 = jnp.ones((128,), jnp.float32)
    zeros640 = jnp.zeros((ROWS_PER_TILE,), jnp.float32)
    zeros_blk = jnp.zeros((ROWS_PER_TILE, F), jnp.float32)

    degp = _deg_sc(dst3, ones128, zeros640)
    degcol = (degp[0] + degp[1]).reshape(NPAD, 1)
    y, dinv = _scale_tc(x_pad, degcol)
    z = _agg_sc(y, src3, dst3, zeros_blk)
    G, m, agg = _stats_tc(z, y, dinv)
    M, c = _finalize_tc(G, m, W1, gamma.reshape(1, H), beta.reshape(1, H),
                        W2, b2.reshape(1, OUT))
    maskcol = jnp.pad(mask.astype(jnp.float32), (0, NPAD - N)).reshape(NPAD, 1)
    out = _head_tc(agg, M, c, maskcol)
    return out[:N]


# speedup vs baseline: 35.8454x; 1.3604x over previous
import functools

import jax
import jax.numpy as jnp
from jax import lax
from jax.experimental import pallas as pl
from jax.experimental.pallas import tpu as pltpu
from jax.experimental.pallas import tpu_sc as plsc

N = 10000
NPAD = 10240
E = 320000
F = 128
H = 512
OUT = 2
NC, NS = 2, 16
NW = NC * NS
EPT = E // NW
CH, CW = 80, 125
ROWS_PER_TILE = NPAD // NS
RB = 512

_MESH = dict(core_axis_name="c", subcore_axis_name="s", num_cores=NC,
             num_subcores=NS)


def _deg_body(dst_hbm, ones_hbm, zeros_hbm, out_hbm, didx_v, ones_v, deg_sh,
              sem):
    c = lax.axis_index("c")
    s = lax.axis_index("s")
    wid = c * NS + s
    pltpu.sync_copy(dst_hbm.at[wid], didx_v)
    pltpu.sync_copy(ones_hbm, ones_v)
    pltpu.sync_copy(zeros_hbm, deg_sh.at[pl.ds(s * ROWS_PER_TILE,
                                               ROWS_PER_TILE)])
    plsc.subcore_barrier()

    def fire(j, carry):
        pltpu.async_copy(ones_v.at[pl.ds(0, CW)], deg_sh.at[didx_v.at[j]],
                         sem, add=True)
        return carry

    lax.fori_loop(0, CH, fire, 0)

    def drain(j, carry):
        pltpu.make_async_copy(ones_v.at[pl.ds(0, CW)],
                              deg_sh.at[didx_v.at[j]], sem).wait()
        return carry

    lax.fori_loop(0, CH, drain, 0)
    plsc.subcore_barrier()
    sl = pl.ds(s * ROWS_PER_TILE, ROWS_PER_TILE)
    pltpu.sync_copy(deg_sh.at[sl], out_hbm.at[c, sl])


def _deg_sc(dst3, ones128, zeros640):
    k = functools.partial(
        pl.kernel,
        out_type=jax.ShapeDtypeStruct((NC, NPAD), jnp.float32),
        mesh=plsc.VectorSubcoreMesh(**_MESH),
        scratch_types=[
            pltpu.VMEM((CH, CW), jnp.int32),
            pltpu.VMEM((128,), jnp.float32),
            pltpu.VMEM_SHARED((NPAD,), jnp.float32),
            pltpu.SemaphoreType.DMA,
        ],
    )(_deg_body)
    return k(dst3, ones128, zeros640)


def _agg_body(y_hbm, src_hbm, dst_hbm, zeros_hbm, out_hbm, sidx_v, didx_v,
              gbuf, z_sh):
    c = lax.axis_index("c")
    s = lax.axis_index("s")
    wid = c * NS + s
    pltpu.sync_copy(src_hbm.at[wid], sidx_v)
    pltpu.sync_copy(dst_hbm.at[wid], didx_v)
    rsl = pl.ds(s * ROWS_PER_TILE, ROWS_PER_TILE)

    @pl.when(c == 0)
    def _():
        pltpu.sync_copy(y_hbm.at[rsl, :], z_sh.at[rsl, :])

    @pl.when(c != 0)
    def _():
        pltpu.sync_copy(zeros_hbm, z_sh.at[rsl, :])

    plsc.subcore_barrier()

    def body(j, carry):
        pltpu.sync_copy(y_hbm.at[sidx_v.at[j]], gbuf)
        pltpu.sync_copy(gbuf, z_sh.at[didx_v.at[j]], add=True)
        return carry

    lax.fori_loop(0, CH, body, 0)
    plsc.subcore_barrier()
    pltpu.sync_copy(z_sh.at[rsl, :], out_hbm.at[c, rsl, :])


def _agg_sc(y, src3, dst3, zeros_blk):
    k = functools.partial(
        pl.kernel,
        out_type=jax.ShapeDtypeStruct((NC, NPAD, F), jnp.float32),
        mesh=plsc.VectorSubcoreMesh(**_MESH),
        scratch_types=[
            pltpu.VMEM((CH, CW), jnp.int32),
            pltpu.VMEM((CH, CW), jnp.int32),
            pltpu.VMEM((CW, F), jnp.float32),
            pltpu.VMEM_SHARED((NPAD, F), jnp.float32),
        ],
    )(_agg_body)
    return k(y, src3, dst3, zeros_blk)


def _scale_body(x_ref, d_ref, y_ref, dinv_ref):
    inv = lax.rsqrt(d_ref[...] + 1.0)
    y_ref[...] = x_ref[...] * inv
    dinv_ref[...] = inv


def _scale_tc(x_pad, degcol):
    return pl.pallas_call(
        _scale_body,
        grid=(NPAD // RB,),
        in_specs=[
            pl.BlockSpec((RB, F), lambda i: (i, 0)),
            pl.BlockSpec((RB, 1), lambda i: (i, 0)),
        ],
        out_specs=[
            pl.BlockSpec((RB, F), lambda i: (i, 0)),
            pl.BlockSpec((RB, 1), lambda i: (i, 0)),
        ],
        out_shape=[
            jax.ShapeDtypeStruct((NPAD, F), jnp.float32),
            jax.ShapeDtypeStruct((NPAD, 1), jnp.float32),
        ],
    )(x_pad, degcol)


def _stats_body(z_ref, dinv_ref, G_ref, m_ref, agg_ref):
    i = pl.program_id(0)
    z = z_ref[...]
    a = (z[0] + z[1]) * dinv_ref[...]
    agg_ref[...] = a

    @pl.when(i == 0)
    def _():
        G_ref[...] = jnp.zeros_like(G_ref)
        m_ref[...] = jnp.zeros_like(m_ref)

    G_ref[...] += lax.dot_general(
        a, a, (((0,), (0,)), ((), ())),
        preferred_element_type=jnp.float32,
        precision=lax.Precision.HIGHEST)
    m_ref[...] += jnp.sum(a, axis=0, keepdims=True)


def _stats_tc(z, dinv):
    return pl.pallas_call(
        _stats_body,
        grid=(NPAD // RB,),
        in_specs=[
            pl.BlockSpec((NC, RB, F), lambda i: (0, i, 0)),
            pl.BlockSpec((RB, 1), lambda i: (i, 0)),
        ],
        out_specs=[
            pl.BlockSpec((F, F), lambda i: (0, 0)),
            pl.BlockSpec((1, F), lambda i: (0, 0)),
            pl.BlockSpec((RB, F), lambda i: (i, 0)),
        ],
        out_shape=[
            jax.ShapeDtypeStruct((F, F), jnp.float32),
            jax.ShapeDtypeStruct((1, F), jnp.float32),
            jax.ShapeDtypeStruct((NPAD, F), jnp.float32),
        ],
    )(z, dinv)


def _finalize_body(G_ref, m_ref, W1_ref, g_ref, be_ref, W2_ref, b2_ref,
                   M_ref, c_ref):
    invn = 1.0 / N
    W1 = W1_ref[...]
    mn = m_ref[...] * invn
    mh0 = lax.dot_general(mn, W1, (((1,), (0,)), ((), ())),
                          preferred_element_type=jnp.float32,
                          precision=lax.Precision.HIGHEST)
    GW = lax.dot_general(G_ref[...] * invn, W1,
                         (((1,), (0,)), ((), ())),
                         preferred_element_type=jnp.float32,
                         precision=lax.Precision.HIGHEST)
    varh = jnp.sum(GW * W1, axis=0, keepdims=True) - mh0 * mh0
    sv = g_ref[...] * lax.rsqrt(varh + 1e-5)
    M_ref[...] = lax.dot_general(W1 * sv, W2_ref[...],
                                 (((1,), (0,)), ((), ())),
                                 preferred_element_type=jnp.float32,
                                 precision=lax.Precision.HIGHEST)
    c_ref[...] = lax.dot_general(be_ref[...] - mh0 * sv, W2_ref[...],
                                 (((1,), (0,)), ((), ())),
                                 preferred_element_type=jnp.float32,
                                 precision=lax.Precision.HIGHEST) + b2_ref[...]


def _finalize_tc(G, m, W1, gamma2, beta2, W2, b22):
    return pl.pallas_call(
        _finalize_body,
        out_shape=[
            jax.ShapeDtypeStruct((F, OUT), jnp.float32),
            jax.ShapeDtypeStruct((1, OUT), jnp.float32),
        ],
    )(G, m, W1, gamma2, beta2, W2, b22)


def _head_body(agg_ref, M_ref, c_ref, mk_ref, o_ref):
    l = lax.dot_general(agg_ref[...], M_ref[...], (((1,), (0,)), ((), ())),
                        preferred_element_type=jnp.float32,
                        precision=lax.Precision.HIGHEST) + c_ref[...]
    r = jnp.maximum(l, 0.0) * mk_ref[...]
    mx = jnp.max(r, axis=1, keepdims=True)
    e = jnp.exp(r - mx)
    o_ref[...] = e / jnp.sum(e, axis=1, keepdims=True)


def _head_tc(agg, M, c, maskcol):
    return pl.pallas_call(
        _head_body,
        grid=(NPAD // RB,),
        in_specs=[
            pl.BlockSpec((RB, F), lambda i: (i, 0)),
            pl.BlockSpec((F, OUT), lambda i: (0, 0)),
            pl.BlockSpec((1, OUT), lambda i: (0, 0)),
            pl.BlockSpec((RB, 1), lambda i: (i, 0)),
        ],
        out_specs=pl.BlockSpec((RB, OUT), lambda i: (i, 0)),
        out_shape=jax.ShapeDtypeStruct((NPAD, OUT), jnp.float32),
    )(agg, M, c, maskcol)


def kernel(state, edge_index, edge_attr, mask, W1, b1, gamma, beta, W2, b2):
    del edge_attr, b1
    x = state.reshape(N, F)
    x_pad = jnp.pad(x, ((0, NPAD - N), (0, 0)))
    src3 = edge_index[0].reshape(NW, CH, CW)
    dst3 = edge_index[1].reshape(NW, CH, CW)
    ones128 = jnp.ones((128,), jnp.float32)
    zeros640 = jnp.zeros((ROWS_PER_TILE,), jnp.float32)
    zeros_blk = jnp.zeros((ROWS_PER_TILE, F), jnp.float32)

    degp = _deg_sc(dst3, ones128, zeros640)
    degcol = (degp[0] + degp[1]).reshape(NPAD, 1)
    y, dinv = _scale_tc(x_pad, degcol)
    z = _agg_sc(y, src3, dst3, zeros_blk)
    G, m, agg = _stats_tc(z, dinv)
    M, c = _finalize_tc(G, m, W1, gamma.reshape(1, H), beta.reshape(1, H),
                        W2, b2.reshape(1, OUT))
    maskcol = jnp.pad(mask.astype(jnp.float32), (0, NPAD - N)).reshape(NPAD, 1)
    out = _head_tc(agg, M, c, maskcol)
    return out[:N]


# speedup vs baseline: 50.7569x; 1.4160x over previous
import functools

import jax
import jax.numpy as jnp
from jax import lax
from jax.experimental import pallas as pl
from jax.experimental.pallas import tpu as pltpu
from jax.experimental.pallas import tpu_sc as plsc

N = 10000
ZPAD = 10112
DEGPAD = 10240
E = 320000
F = 128
H = 512
OUT = 2
NC, NS = 2, 16
NW = NC * NS
EPT = E // NW
CH, CW = 80, 125
CHH = 40
ROWS_PER_TILE = ZPAD // NS
DEG_ROWS_PER_TILE = DEGPAD // NS
RB = 1264

_MESH = dict(core_axis_name="c", subcore_axis_name="s", num_cores=NC,
             num_subcores=NS)


def _deg_body(dst_hbm, ones_hbm, zeros_hbm, out_hbm, didx_v, ones_v, deg_sh,
              sem):
    c = lax.axis_index("c")
    s = lax.axis_index("s")
    wid = c * NS + s
    pltpu.sync_copy(dst_hbm.at[wid], didx_v)
    pltpu.sync_copy(ones_hbm, ones_v)
    pltpu.sync_copy(zeros_hbm, deg_sh.at[pl.ds(s * DEG_ROWS_PER_TILE,
                                               DEG_ROWS_PER_TILE)])
    plsc.subcore_barrier()

    def fire(j, carry):
        pltpu.async_copy(ones_v.at[pl.ds(0, CW)], deg_sh.at[didx_v.at[j]],
                         sem, add=True)
        return carry

    lax.fori_loop(0, CH, fire, 0)

    def drain(j, carry):
        pltpu.make_async_copy(ones_v.at[pl.ds(0, CW)],
                              deg_sh.at[didx_v.at[j]], sem).wait()
        return carry

    lax.fori_loop(0, CH, drain, 0)
    plsc.subcore_barrier()
    sl = pl.ds(s * DEG_ROWS_PER_TILE, DEG_ROWS_PER_TILE)
    pltpu.sync_copy(deg_sh.at[sl], out_hbm.at[c, sl])


def _deg_sc(dst3, ones128, zeros640):
    k = functools.partial(
        pl.kernel,
        out_type=jax.ShapeDtypeStruct((NC, DEGPAD), jnp.float32),
        mesh=plsc.VectorSubcoreMesh(**_MESH),
        scratch_types=[
            pltpu.VMEM((CH, CW), jnp.int32),
            pltpu.VMEM((128,), jnp.float32),
            pltpu.VMEM_SHARED((DEGPAD,), jnp.float32),
            pltpu.SemaphoreType.DMA,
        ],
    )(_deg_body)
    return k(dst3, ones128, zeros640)


def _agg_body(y_hbm, src_hbm, dst_hbm, zeros_hbm, out_hbm, sidx_v, didx_v,
              gbuf, z_sh, gsem):
    c = lax.axis_index("c")
    s = lax.axis_index("s")
    wid = c * NS + s
    rsl = pl.ds(s * ROWS_PER_TILE, ROWS_PER_TILE)

    @pl.when(c == 0)
    def _():
        pltpu.sync_copy(y_hbm.at[rsl, :], z_sh.at[rsl, :])

    @pl.when(c != 0)
    def _():
        pltpu.sync_copy(zeros_hbm, z_sh.at[rsl, :])

    plsc.subcore_barrier()

    for h in range(CH // CHH):
        pltpu.sync_copy(src_hbm.at[wid, pl.ds(h * CHH, CHH)], sidx_v)
        pltpu.sync_copy(dst_hbm.at[wid, pl.ds(h * CHH, CHH)], didx_v)
        pltpu.async_copy(y_hbm.at[sidx_v.at[0]], gbuf.at[0], gsem.at[0])
        pltpu.async_copy(y_hbm.at[sidx_v.at[1]], gbuf.at[1], gsem.at[1])

        def body(j, carry):
            slot = lax.rem(j, 2)
            pltpu.make_async_copy(y_hbm.at[sidx_v.at[j]], gbuf.at[slot],
                                  gsem.at[slot]).wait()
            pltpu.sync_copy(gbuf.at[slot], z_sh.at[didx_v.at[j]], add=True)

            @pl.when(j + 2 < CHH)
            def _():
                pltpu.async_copy(y_hbm.at[sidx_v.at[j + 2]], gbuf.at[slot],
                                 gsem.at[slot])

            return carry

        lax.fori_loop(0, CHH, body, 0)

    plsc.subcore_barrier()
    pltpu.sync_copy(z_sh.at[rsl, :], out_hbm.at[c, rsl, :])


def _agg_sc(y, src3, dst3, zeros_blk):
    k = functools.partial(
        pl.kernel,
        out_type=jax.ShapeDtypeStruct((NC, ZPAD, F), jnp.float32),
        mesh=plsc.VectorSubcoreMesh(**_MESH),
        scratch_types=[
            pltpu.VMEM((CHH, CW), jnp.int32),
            pltpu.VMEM((CHH, CW), jnp.int32),
            pltpu.VMEM((2, CW, F), jnp.float32),
            pltpu.VMEM_SHARED((ZPAD, F), jnp.float32),
            pltpu.SemaphoreType.DMA((2,)),
        ],
    )(_agg_body)
    return k(y, src3, dst3, zeros_blk)


def _scale_body(x_ref, d_ref, y_ref, dinv_ref):
    inv = lax.rsqrt(d_ref[...] + 1.0)
    y_ref[...] = x_ref[...] * inv
    dinv_ref[...] = inv


def _scale_tc(x_pad, degcol):
    return pl.pallas_call(
        _scale_body,
        grid=(ZPAD // RB,),
        in_specs=[
            pl.BlockSpec((RB, F), lambda i: (i, 0)),
            pl.BlockSpec((RB, 1), lambda i: (i, 0)),
        ],
        out_specs=[
            pl.BlockSpec((RB, F), lambda i: (i, 0)),
            pl.BlockSpec((RB, 1), lambda i: (i, 0)),
        ],
        out_shape=[
            jax.ShapeDtypeStruct((ZPAD, F), jnp.float32),
            jax.ShapeDtypeStruct((ZPAD, 1), jnp.float32),
        ],
    )(x_pad, degcol)


def _stats_body(z_ref, dinv_ref, G_ref, m_ref, agg_ref):
    i = pl.program_id(0)
    z = z_ref[...]
    a = (z[0] + z[1]) * dinv_ref[...]
    agg_ref[...] = a

    @pl.when(i == 0)
    def _():
        G_ref[...] = jnp.zeros_like(G_ref)
        m_ref[...] = jnp.zeros_like(m_ref)

    G_ref[...] += lax.dot_general(
        a, a, (((0,), (0,)), ((), ())),
        preferred_element_type=jnp.float32,
        precision=lax.Precision.HIGHEST)
    m_ref[...] += jnp.sum(a, axis=0, keepdims=True)


def _stats_tc(z, dinv):
    return pl.pallas_call(
        _stats_body,
        grid=(ZPAD // RB,),
        in_specs=[
            pl.BlockSpec((NC, RB, F), lambda i: (0, i, 0)),
            pl.BlockSpec((RB, 1), lambda i: (i, 0)),
        ],
        out_specs=[
            pl.BlockSpec((F, F), lambda i: (0, 0)),
            pl.BlockSpec((1, F), lambda i: (0, 0)),
            pl.BlockSpec((RB, F), lambda i: (i, 0)),
        ],
        out_shape=[
            jax.ShapeDtypeStruct((F, F), jnp.float32),
            jax.ShapeDtypeStruct((1, F), jnp.float32),
            jax.ShapeDtypeStruct((ZPAD, F), jnp.float32),
        ],
    )(z, dinv)


def _finalize_body(G_ref, m_ref, W1_ref, g_ref, be_ref, W2_ref, b2_ref,
                   M_ref, c_ref):
    invn = 1.0 / N
    W1 = W1_ref[...]
    mn = m_ref[...] * invn
    mh0 = lax.dot_general(mn, W1, (((1,), (0,)), ((), ())),
                          preferred_element_type=jnp.float32,
                          precision=lax.Precision.HIGHEST)
    GW = lax.dot_general(G_ref[...] * invn, W1,
                         (((1,), (0,)), ((), ())),
                         preferred_element_type=jnp.float32,
                         precision=lax.Precision.HIGHEST)
    varh = jnp.sum(GW * W1, axis=0, keepdims=True) - mh0 * mh0
    sv = g_ref[...] * lax.rsqrt(varh + 1e-5)
    M_ref[...] = lax.dot_general(W1 * sv, W2_ref[...],
                                 (((1,), (0,)), ((), ())),
                                 preferred_element_type=jnp.float32,
                                 precision=lax.Precision.HIGHEST)
    c_ref[...] = lax.dot_general(be_ref[...] - mh0 * sv, W2_ref[...],
                                 (((1,), (0,)), ((), ())),
                                 preferred_element_type=jnp.float32,
                                 precision=lax.Precision.HIGHEST) + b2_ref[...]


def _finalize_tc(G, m, W1, gamma2, beta2, W2, b22):
    return pl.pallas_call(
        _finalize_body,
        out_shape=[
            jax.ShapeDtypeStruct((F, OUT), jnp.float32),
            jax.ShapeDtypeStruct((1, OUT), jnp.float32),
        ],
    )(G, m, W1, gamma2, beta2, W2, b22)


def _head_body(agg_ref, M_ref, c_ref, mk_ref, o_ref):
    l = lax.dot_general(agg_ref[...], M_ref[...], (((1,), (0,)), ((), ())),
                        preferred_element_type=jnp.float32,
                        precision=lax.Precision.HIGHEST) + c_ref[...]
    r = jnp.maximum(l, 0.0) * mk_ref[...]
    mx = jnp.max(r, axis=1, keepdims=True)
    e = jnp.exp(r - mx)
    o_ref[...] = e / jnp.sum(e, axis=1, keepdims=True)


def _head_tc(agg, M, c, maskcol):
    return pl.pallas_call(
        _head_body,
        grid=(ZPAD // RB,),
        in_specs=[
            pl.BlockSpec((RB, F), lambda i: (i, 0)),
            pl.BlockSpec((F, OUT), lambda i: (0, 0)),
            pl.BlockSpec((1, OUT), lambda i: (0, 0)),
            pl.BlockSpec((RB, 1), lambda i: (i, 0)),
        ],
        out_specs=pl.BlockSpec((RB, OUT), lambda i: (i, 0)),
        out_shape=jax.ShapeDtypeStruct((ZPAD, OUT), jnp.float32),
    )(agg, M, c, maskcol)


def kernel(state, edge_index, edge_attr, mask, W1, b1, gamma, beta, W2, b2):
    del edge_attr, b1
    x = state.reshape(N, F)
    x_pad = jnp.pad(x, ((0, ZPAD - N), (0, 0)))
    src3 = edge_index[0].reshape(NW, CH, CW)
    dst3 = edge_index[1].reshape(NW, CH, CW)
    ones128 = jnp.ones((128,), jnp.float32)
    zeros640 = jnp.zeros((DEG_ROWS_PER_TILE,), jnp.float32)
    zeros_blk = jnp.zeros((ROWS_PER_TILE, F), jnp.float32)

    degp = _deg_sc(dst3, ones128, zeros640)
    degcol = (degp[0, :ZPAD] + degp[1, :ZPAD]).reshape(ZPAD, 1)
    y, dinv = _scale_tc(x_pad, degcol)
    z = _agg_sc(y, src3, dst3, zeros_blk)
    G, m, agg = _stats_tc(z, dinv)
    M, c = _finalize_tc(G, m, W1, gamma.reshape(1, H), beta.reshape(1, H),
                        W2, b2.reshape(1, OUT))
    maskcol = jnp.pad(mask.astype(jnp.float32), (0, ZPAD - N)).reshape(ZPAD, 1)
    return _head_tc(agg, M, c, maskcol)[:N]
